# baseline (speedup 1.0000x reference)
import functools

import jax
import jax.numpy as jnp
from jax import lax
from jax.experimental import pallas as pl
from jax.experimental.pallas import tpu as pltpu
from jax.experimental.pallas import tpu_sc as plsc

N = 8192
D = 64
B = 4608
BR = 1152
G = B // BR
CH = 2048
NCH = N // CH
NG = CH // 128
NW = 32
BPW = B // NW
CPT = N // NW
HALF = BPW // 2
DP = 128
LUTR = 40


def _argmin_kernel(x_ref, e_ref, idx_ref, epad_ref, lut_ref):
    x = x_ref[...]
    a = jnp.sum(x ** 2, axis=1, keepdims=True)
    rv = jnp.full((BR, 128), jnp.inf, jnp.float32)
    rg = jnp.zeros((BR, 128), jnp.int32)
    for c in reversed(range(NCH)):
        e = e_ref[c * CH:(c + 1) * CH, :]
        b = jnp.sum(e ** 2, axis=1)
        prod2 = lax.dot_general(x, e + e, (((1,), (1,)), ((), ())),
                                preferred_element_type=jnp.float32)
        for g in reversed(range(NG)):
            dsub = (a + b[g * 128:(g + 1) * 128][None, :]) \
                - prod2[:, g * 128:(g + 1) * 128]
            upd = dsub <= rv
            rg = jnp.where(upd, c * NG + g, rg)
            rv = jnp.minimum(dsub, rv)
    gidx = rg * 128 + lax.broadcasted_iota(jnp.int32, (BR, 128), 1)
    gmin = jnp.min(rv, axis=1)
    cand = jnp.where(rv == gmin[:, None], gidx, N)
    idx_ref[0, 0, :] = jnp.min(cand, axis=1)

    @pl.when(pl.program_id(0) == 0)
    def _():
        epad_ref[:, :D] = e_ref[...]
        epad_ref[:, D:] = jnp.zeros((N, DP - D), jnp.float32)
        k = (lax.broadcasted_iota(jnp.int32, (LUTR, 128), 0) * 128
             + lax.broadcasted_iota(jnp.int32, (LUTR, 128), 1))
        p = k.astype(jnp.float32) / float(B)
        lut_ref[...] = p * jnp.log(p + 1e-5)


_argmin_call = pl.pallas_call(
    _argmin_kernel,
    grid=(G,),
    in_specs=[pl.BlockSpec((BR, D), lambda i: (i, 0)),
              pl.BlockSpec((N, D), lambda i: (0, 0))],
    out_specs=[pl.BlockSpec((1, 1, BR), lambda i: (i, 0, 0)),
               pl.BlockSpec((N, DP), lambda i: (0, 0)),
               pl.BlockSpec((LUTR, 128), lambda i: (0, 0))],
    out_shape=[jax.ShapeDtypeStruct((G, 1, BR), jnp.int32),
               jax.ShapeDtypeStruct((N, DP), jnp.float32),
               jax.ShapeDtypeStruct((LUTR, 128), jnp.float32)],
)


@functools.cache
def _make_sc_fused():
    mesh = plsc.VectorSubcoreMesh(core_axis_name="c", subcore_axis_name="s")
    num_cores = mesh.num_cores

    @functools.partial(
        pl.kernel,
        mesh=mesh,
        out_type=[jax.ShapeDtypeStruct((B, D), jnp.float32),
                  jax.ShapeDtypeStruct((N,), jnp.float32),
                  jax.ShapeDtypeStruct((16 * NW,), jnp.float32),
                  jax.ShapeDtypeStruct((16 * NW,), jnp.float32)],
        scratch_types=[pltpu.VMEM((HALF,), jnp.int32),
                       pltpu.VMEM((HALF,), jnp.int32),
                       pltpu.VMEM((HALF, DP), jnp.float32),
                       pltpu.VMEM((HALF, DP), jnp.float32),
                       pltpu.VMEM((BPW, D), jnp.float32),
                       pltpu.VMEM((BPW, D), jnp.float32),
                       pltpu.VMEM((B,), jnp.int32),
                       pltpu.VMEM((CPT,), jnp.float32),
                       pltpu.VMEM((CPT,), jnp.float32),
                       pltpu.VMEM((LUTR, 128), jnp.float32),
                       pltpu.VMEM((16,), jnp.float32),
                       pltpu.VMEM((16,), jnp.float32),
                       pltpu.SemaphoreType.DMA],
        compiler_params=pltpu.CompilerParams(needs_layout_passes=False),
    )
    def _sc_fused(emb_hbm, idx_hbm, x_hbm, lut_hbm,
                  quant_hbm, avgp_hbm, msep_hbm, entp_hbm,
                  idx_a, idx_b, rows_a, rows_b, x_v, qout_v,
                  idx_all, cnt_v, p_v, lut_v, part_v, entv_v, sem):
        wid = lax.axis_index("s") * num_cores + lax.axis_index("c")
        base = wid * BPW
        lo = wid * CPT
        pltpu.sync_copy(idx_hbm.at[pl.ds(base, HALF)], idx_a)
        pltpu.sync_copy(idx_hbm.at[pl.ds(base + HALF, HALF)], idx_b)
        ca = pltpu.async_copy(emb_hbm.at[idx_a], rows_a, sem)
        cb = pltpu.async_copy(emb_hbm.at[idx_b], rows_b, sem)
        pltpu.sync_copy(x_hbm.at[pl.ds(base, BPW)], x_v)
        pltpu.sync_copy(lut_hbm, lut_v)
        pltpu.sync_copy(idx_hbm, idx_all)
        ca.wait()
        cb.wait()

        def _half(rows_ref, roff):
            def body(r, acc):
                for k in range(D // 16):
                    q16 = rows_ref[r, pl.ds(k * 16, 16)]
                    x16 = x_v[roff + r, pl.ds(k * 16, 16)]
                    df = q16 - x16
                    qout_v[roff + r, pl.ds(k * 16, 16)] = x16 + df
                    acc = acc + df * df
                return acc
            return body

        acc = jnp.zeros((16,), jnp.float32)
        acc = lax.fori_loop(0, HALF, _half(rows_a, 0), acc)
        acc = lax.fori_loop(0, HALF, _half(rows_b, HALF), acc)
        part_v[...] = acc
        pltpu.sync_copy(qout_v, quant_hbm.at[pl.ds(base, BPW)])
        pltpu.sync_copy(part_v, msep_hbm.at[pl.ds(wid * 16, 16)])

        zeros16 = jnp.zeros((16,), jnp.float32)
        for k in range(CPT // 16):
            cnt_v[pl.ds(k * 16, 16)] = zeros16
        ones16 = jnp.ones((16,), jnp.float32)
        UNROLL = 8

        def hbody(i, carry):
            for u in range(UNROLL):
                v = idx_all[pl.ds((i * UNROLL + u) * 16, 16)]
                rel = v - lo
                msk = (rel >= 0) & (rel < CPT)
                relc = jnp.clip(rel, 0, CPT - 1)
                plsc.addupdate_scatter(cnt_v, [relc], ones16, mask=msk)
            return carry

        lax.fori_loop(0, B // (16 * UNROLL), hbody, 0)

        eacc = jnp.zeros((16,), jnp.float32)
        for k in range(CPT // 16):
            c16 = cnt_v[pl.ds(k * 16, 16)]
            p_v[pl.ds(k * 16, 16)] = c16 / float(B)
            ki = c16.astype(jnp.int32)
            hi = ki >> 7
            lo16 = ki & 127
            eacc = eacc + plsc.load_gather(lut_v, [hi, lo16])
        entv_v[...] = eacc
        pltpu.sync_copy(p_v, avgp_hbm.at[pl.ds(lo, CPT)])
        pltpu.sync_copy(entv_v, entp_hbm.at[pl.ds(wid * 16, 16)])

    return _sc_fused


def _scalar_kernel(msep_ref, entp_ref, scal_ref):
    mse = jnp.sum(msep_ref[...]) / float(B * D)
    ent = jnp.sum(entp_ref[...])
    perp = jnp.exp(-ent)
    vq = (mse + 0.25 * mse) + 0.1 * ent
    scal_ref[0:1, :] = jnp.full((1, 128), mse, jnp.float32)
    scal_ref[1:2, :] = jnp.full((1, 128), ent, jnp.float32)
    scal_ref[2:3, :] = jnp.full((1, 128), perp, jnp.float32)
    scal_ref[3:4, :] = jnp.full((1, 128), vq, jnp.float32)
    scal_ref[4:8, :] = jnp.zeros((4, 128), jnp.float32)


_scalar_call = pl.pallas_call(
    _scalar_kernel,
    out_shape=jax.ShapeDtypeStruct((8, 128), jnp.float32),
)


def kernel(x, embedding):
    input_shape = x.shape
    flat_x = x.reshape(-1, D)
    idx3, emb_pad, lut = _argmin_call(flat_x, embedding)
    idx = idx3.reshape(-1)
    quant, avgp, msep, entp = _make_sc_fused()(emb_pad, idx, flat_x, lut)
    scal = _scalar_call(msep, entp)
    quantized = quant.reshape(input_shape)
    mse = scal[0, 0]
    ent = scal[1, 0]
    perp = scal[2, 0]
    vq = scal[3, 0]
    enc_idx = idx.reshape(input_shape[:-1])
    return (quantized, vq, mse, mse, perp, ent, enc_idx, avgp)


# speedup vs baseline: 1.0010x; 1.0010x over previous
import functools

import jax
import jax.numpy as jnp
from jax import lax
from jax.experimental import pallas as pl
from jax.experimental.pallas import tpu as pltpu
from jax.experimental.pallas import tpu_sc as plsc

N = 8192
D = 64
B = 4608
BR = 1152
G = B // BR
CH = 2048
NCH = N // CH
NG = CH // 128
NW = 32
BPW = B // NW
CPT = N // NW
HALF = BPW // 2
DP = 128


def _argmin_kernel(x_ref, e_ref, idx_ref, epad_ref, b_ref):
    @pl.when(pl.program_id(0) == 0)
    def _():
        b_ref[...] = jnp.sum(e_ref[...] ** 2, axis=1).reshape(NCH * NG, 128)

    x = x_ref[...]
    a = jnp.sum(x ** 2, axis=1, keepdims=True)
    x2 = x + x
    RB = 64
    NRB = BR // RB
    rvs = [jnp.full((RB, 128), jnp.inf, jnp.float32) for _ in range(NRB)]
    rgs = [jnp.zeros((RB, 128), jnp.int32) for _ in range(NRB)]
    abs_ = [jnp.broadcast_to(a[rb * RB:(rb + 1) * RB], (RB, 128))
            for rb in range(NRB)]
    for c in reversed(range(NCH)):
        e = e_ref[c * CH:(c + 1) * CH, :]
        prod2 = lax.dot_general(x2, e, (((1,), (1,)), ((), ())),
                                preferred_element_type=jnp.float32)
        for rb in range(NRB):
            rv, rg = rvs[rb], rgs[rb]
            ab = abs_[rb]
            for g in reversed(range(NG)):
                bs = b_ref[c * NG + g:c * NG + g + 1, :]
                dsub = (ab + bs) \
                    - prod2[rb * RB:(rb + 1) * RB, g * 128:(g + 1) * 128]
                upd = dsub <= rv
                rg = jnp.where(upd, c * NG + g, rg)
                rv = jnp.minimum(dsub, rv)
            rvs[rb], rgs[rb] = rv, rg
    lane = lax.broadcasted_iota(jnp.int32, (RB, 128), 1)
    for rb in range(NRB):
        gidx = rgs[rb] * 128 + lane
        gmin = jnp.min(rvs[rb], axis=1)
        cand = jnp.where(rvs[rb] == gmin[:, None], gidx, N)
        idx_ref[0, 0, rb * RB:(rb + 1) * RB] = jnp.min(cand, axis=1)

    @pl.when(pl.program_id(0) == 0)
    def _():
        epad_ref[:, :D] = e_ref[...]
        epad_ref[:, D:] = jnp.zeros((N, DP - D), jnp.float32)


_argmin_call = pl.pallas_call(
    _argmin_kernel,
    grid=(G,),
    in_specs=[pl.BlockSpec((BR, D), lambda i: (i, 0)),
              pl.BlockSpec((N, D), lambda i: (0, 0))],
    out_specs=[pl.BlockSpec((1, 1, BR), lambda i: (i, 0, 0)),
               pl.BlockSpec((N, DP), lambda i: (0, 0))],
    out_shape=[jax.ShapeDtypeStruct((G, 1, BR), jnp.int32),
               jax.ShapeDtypeStruct((N, DP), jnp.float32)],
    scratch_shapes=[pltpu.VMEM((NCH * NG, 128), jnp.float32)],
)


@functools.cache
def _make_sc_gather_hist():
    mesh = plsc.VectorSubcoreMesh(core_axis_name="c", subcore_axis_name="s")
    num_cores = mesh.num_cores

    @functools.partial(
        pl.kernel,
        mesh=mesh,
        out_type=[jax.ShapeDtypeStruct((B, DP), jnp.float32),
                  jax.ShapeDtypeStruct((N,), jnp.float32)],
        scratch_types=[pltpu.VMEM((HALF,), jnp.int32),
                       pltpu.VMEM((HALF,), jnp.int32),
                       pltpu.VMEM((HALF, DP), jnp.float32),
                       pltpu.VMEM((HALF, DP), jnp.float32),
                       pltpu.VMEM((B,), jnp.int32),
                       pltpu.VMEM((CPT,), jnp.float32),
                       pltpu.SemaphoreType.DMA],
        compiler_params=pltpu.CompilerParams(needs_layout_passes=False),
    )
    def _sc_gather_hist(emb_hbm, idx_hbm, q_hbm, cnt_hbm,
                        idx_a, idx_b, rows_a, rows_b, idx_all, cnt_v, sem):
        wid = lax.axis_index("s") * num_cores + lax.axis_index("c")
        base = wid * BPW
        pltpu.sync_copy(idx_hbm.at[pl.ds(base, HALF)], idx_a)
        pltpu.sync_copy(idx_hbm.at[pl.ds(base + HALF, HALF)], idx_b)
        ca = pltpu.async_copy(emb_hbm.at[idx_a], rows_a, sem)
        cb = pltpu.async_copy(emb_hbm.at[idx_b], rows_b, sem)
        pltpu.sync_copy(idx_hbm, idx_all)
        ca.wait()
        cb.wait()
        pltpu.sync_copy(rows_a, q_hbm.at[pl.ds(base, HALF)])
        pltpu.sync_copy(rows_b, q_hbm.at[pl.ds(base + HALF, HALF)])
        lo = wid * CPT
        zeros16 = jnp.zeros((16,), jnp.float32)
        for k in range(CPT // 16):
            cnt_v[pl.ds(k * 16, 16)] = zeros16
        ones16 = jnp.ones((16,), jnp.float32)
        UNROLL = 8

        def body(i, carry):
            for u in range(UNROLL):
                v = idx_all[pl.ds((i * UNROLL + u) * 16, 16)]
                rel = v - lo
                msk = (rel >= 0) & (rel < CPT)
                relc = jnp.clip(rel, 0, CPT - 1)
                plsc.addupdate_scatter(cnt_v, [relc], ones16, mask=msk)
            return carry

        lax.fori_loop(0, B // (16 * UNROLL), body, 0)
        pltpu.sync_copy(cnt_v, cnt_hbm.at[pl.ds(lo, CPT)])

    return _sc_gather_hist


def _loss_kernel(x_ref, q_ref, cnt_ref, quant_ref, avgp_ref, scal_ref):
    xv = x_ref[...]
    qv = q_ref[...]
    diff = qv - xv
    quant_ref[...] = xv + diff
    mse = jnp.sum(diff ** 2) / float(B * D)
    p = cnt_ref[...] / float(B)
    avgp_ref[...] = p
    ent = jnp.sum(p * jnp.log(p + 1e-5))
    perp = jnp.exp(-ent)
    vq = (mse + 0.25 * mse) + 0.1 * ent
    scal_ref[0:1, :] = jnp.full((1, 128), mse, jnp.float32)
    scal_ref[1:2, :] = jnp.full((1, 128), ent, jnp.float32)
    scal_ref[2:3, :] = jnp.full((1, 128), perp, jnp.float32)
    scal_ref[3:4, :] = jnp.full((1, 128), vq, jnp.float32)
    scal_ref[4:8, :] = jnp.zeros((4, 128), jnp.float32)


_loss_call = pl.pallas_call(
    _loss_kernel,
    grid=(1,),
    in_specs=[pl.BlockSpec((B, D), lambda i: (0, 0)),
              pl.BlockSpec((B, D), lambda i: (0, 0)),
              pl.BlockSpec((N,), lambda i: (0,))],
    out_specs=[pl.BlockSpec((B, D), lambda i: (0, 0)),
               pl.BlockSpec((N,), lambda i: (0,)),
               pl.BlockSpec((8, 128), lambda i: (0, 0))],
    out_shape=[jax.ShapeDtypeStruct((B, D), jnp.float32),
               jax.ShapeDtypeStruct((N,), jnp.float32),
               jax.ShapeDtypeStruct((8, 128), jnp.float32)],
)


def kernel(x, embedding):
    input_shape = x.shape
    flat_x = x.reshape(-1, D)
    idx3, emb_pad = _argmin_call(flat_x, embedding)
    idx = idx3.reshape(-1)
    q_pad, counts = _make_sc_gather_hist()(emb_pad, idx)
    quant, avgp, scal = _loss_call(flat_x, q_pad[:, :D], counts)
    quantized = quant.reshape(input_shape)
    mse = scal[0, 0]
    ent = scal[1, 0]
    perp = scal[2, 0]
    vq = scal[3, 0]
    enc_idx = idx.reshape(input_shape[:-1])
    return (quantized, vq, mse, mse, perp, ent, enc_idx, avgp)


# speedup vs baseline: 1.0185x; 1.0175x over previous
import functools

import jax
import jax.numpy as jnp
from jax import lax
from jax.experimental import pallas as pl
from jax.experimental.pallas import tpu as pltpu
from jax.experimental.pallas import tpu_sc as plsc

N = 8192
D = 64
B = 4608
BR = 1152
G = B // BR
CH = 2048
NCH = N // CH
NG = CH // 128
NW = 32
BPW = B // NW
CPT = N // NW
HALF = BPW // 2
DP = 128


def _argmin_kernel(x_ref, e_ref, idx_ref, epad_ref):
    x = x_ref[...]
    a = jnp.sum(x ** 2, axis=1, keepdims=True)
    rv = jnp.full((BR, 128), jnp.inf, jnp.float32)
    rg = jnp.zeros((BR, 128), jnp.int32)
    for c in range(NCH):
        e = e_ref[c * CH:(c + 1) * CH, :]
        b = jnp.sum(e ** 2, axis=1)
        prod2 = lax.dot_general(x, e + e, (((1,), (1,)), ((), ())),
                                preferred_element_type=jnp.float32)
        for g in range(NG):
            dsub = (a + b[g * 128:(g + 1) * 128][None, :]) \
                - prod2[:, g * 128:(g + 1) * 128]
            upd = dsub < rv
            rv = jnp.where(upd, dsub, rv)
            rg = jnp.where(upd, c * NG + g, rg)
    gidx = rg * 128 + lax.broadcasted_iota(jnp.int32, (BR, 128), 1)
    gmin = jnp.min(rv, axis=1)
    cand = jnp.where(rv == gmin[:, None], gidx, N)
    idx_ref[0, 0, :] = jnp.min(cand, axis=1)

    @pl.when(pl.program_id(0) == 0)
    def _():
        epad_ref[:, :D] = e_ref[...]
        epad_ref[:, D:] = jnp.zeros((N, DP - D), jnp.float32)


_argmin_call = pl.pallas_call(
    _argmin_kernel,
    grid=(G,),
    in_specs=[pl.BlockSpec((BR, D), lambda i: (i, 0)),
              pl.BlockSpec((N, D), lambda i: (0, 0))],
    out_specs=[pl.BlockSpec((1, 1, BR), lambda i: (i, 0, 0)),
               pl.BlockSpec((N, DP), lambda i: (0, 0))],
    out_shape=[jax.ShapeDtypeStruct((G, 1, BR), jnp.int32),
               jax.ShapeDtypeStruct((N, DP), jnp.float32)],
)


@functools.cache
def _make_sc_gather_hist():
    mesh = plsc.VectorSubcoreMesh(core_axis_name="c", subcore_axis_name="s")
    num_cores = mesh.num_cores

    @functools.partial(
        pl.kernel,
        mesh=mesh,
        out_type=[jax.ShapeDtypeStruct((B, DP), jnp.float32),
                  jax.ShapeDtypeStruct((N,), jnp.float32)],
        scratch_types=[pltpu.VMEM((HALF,), jnp.int32),
                       pltpu.VMEM((HALF,), jnp.int32),
                       pltpu.VMEM((HALF, DP), jnp.float32),
                       pltpu.VMEM((HALF, DP), jnp.float32),
                       pltpu.VMEM((B,), jnp.int32),
                       pltpu.VMEM((CPT,), jnp.float32),
                       pltpu.SemaphoreType.DMA],
        compiler_params=pltpu.CompilerParams(needs_layout_passes=False),
    )
    def _sc_gather_hist(emb_hbm, idx_hbm, q_hbm, cnt_hbm,
                        idx_a, idx_b, rows_a, rows_b, idx_all, cnt_v, sem):
        wid = lax.axis_index("s") * num_cores + lax.axis_index("c")
        base = wid * BPW
        pltpu.sync_copy(idx_hbm.at[pl.ds(base, HALF)], idx_a)
        pltpu.sync_copy(idx_hbm.at[pl.ds(base + HALF, HALF)], idx_b)
        ca = pltpu.async_copy(emb_hbm.at[idx_a], rows_a, sem)
        cb = pltpu.async_copy(emb_hbm.at[idx_b], rows_b, sem)
        pltpu.sync_copy(idx_hbm, idx_all)
        ca.wait()
        cb.wait()
        pltpu.sync_copy(rows_a, q_hbm.at[pl.ds(base, HALF)])
        pltpu.sync_copy(rows_b, q_hbm.at[pl.ds(base + HALF, HALF)])
        lo = wid * CPT
        zeros16 = jnp.zeros((16,), jnp.float32)
        for k in range(CPT // 16):
            cnt_v[pl.ds(k * 16, 16)] = zeros16
        ones16 = jnp.ones((16,), jnp.float32)
        UNROLL = 8

        def body(i, carry):
            for u in range(UNROLL):
                v = idx_all[pl.ds((i * UNROLL + u) * 16, 16)]
                rel = v - lo
                msk = (rel >= 0) & (rel < CPT)
                relc = jnp.clip(rel, 0, CPT - 1)
                plsc.addupdate_scatter(cnt_v, [relc], ones16, mask=msk)
            return carry

        lax.fori_loop(0, B // (16 * UNROLL), body, 0)
        pltpu.sync_copy(cnt_v, cnt_hbm.at[pl.ds(lo, CPT)])

    return _sc_gather_hist


def _loss_kernel(x_ref, q_ref, cnt_ref, quant_ref, avgp_ref, scal_ref):
    xv = x_ref[...]
    qv = q_ref[...]
    diff = qv - xv
    quant_ref[...] = xv + diff
    mse = jnp.sum(diff ** 2) / float(B * D)
    p = cnt_ref[...] / float(B)
    avgp_ref[...] = p
    ent = jnp.sum(p * jnp.log(p + 1e-5))
    perp = jnp.exp(-ent)
    vq = (mse + 0.25 * mse) + 0.1 * ent
    scal_ref[0:1, :] = jnp.full((1, 128), mse, jnp.float32)
    scal_ref[1:2, :] = jnp.full((1, 128), ent, jnp.float32)
    scal_ref[2:3, :] = jnp.full((1, 128), perp, jnp.float32)
    scal_ref[3:4, :] = jnp.full((1, 128), vq, jnp.float32)
    scal_ref[4:8, :] = jnp.zeros((4, 128), jnp.float32)


_loss_call = pl.pallas_call(
    _loss_kernel,
    grid=(1,),
    in_specs=[pl.BlockSpec((B, D), lambda i: (0, 0)),
              pl.BlockSpec((B, D), lambda i: (0, 0)),
              pl.BlockSpec((N,), lambda i: (0,))],
    out_specs=[pl.BlockSpec((B, D), lambda i: (0, 0)),
               pl.BlockSpec((N,), lambda i: (0,)),
               pl.BlockSpec((8, 128), lambda i: (0, 0))],
    out_shape=[jax.ShapeDtypeStruct((B, D), jnp.float32),
               jax.ShapeDtypeStruct((N,), jnp.float32),
               jax.ShapeDtypeStruct((8, 128), jnp.float32)],
)


def kernel(x, embedding):
    input_shape = x.shape
    flat_x = x.reshape(-1, D)
    idx3, emb_pad = _argmin_call(flat_x, embedding)
    idx = idx3.reshape(-1)
    q_pad, counts = _make_sc_gather_hist()(emb_pad, idx)
    quant, avgp, scal = _loss_call(flat_x, q_pad[:, :D], counts)
    quantized = quant.reshape(input_shape)
    mse = scal[0, 0]
    ent = scal[1, 0]
    perp = scal[2, 0]
    vq = scal[3, 0]
    enc_idx = idx.reshape(input_shape[:-1])
    return (quantized, vq, mse, mse, perp, ent, enc_idx, avgp)
